# baseline (speedup 1.0000x reference)
import functools

import numpy as np
import jax
import jax.numpy as jnp
from jax import lax
from jax.experimental import pallas as pl
from jax.experimental.pallas import tpu as pltpu
from jax.experimental.pallas import tpu_sc as plsc

_F = 26
_D = 16
_FIELD_DIM = 10000
_ROWS_PER_TABLE = _F * _FIELD_DIM
_NROW = _F * _ROWS_PER_TABLE
_B = 4096
_NPAIR = (_F * (_F - 1)) // 2
_NSLOT = 2 * _NPAIR
_NSLOT_PAD = 656
_CHUNKS = [(0, 128), (128, 128), (256, 128), (384, 128), (512, 128), (640, 16)]

_RB_FULL = _ROWS_PER_TABLE // 128
_R_TAIL = _RB_FULL * 128
_W_TAIL = _ROWS_PER_TABLE - _R_TAIL

_off_np = np.zeros((_NSLOT_PAD,), np.int32)
_fld_np = np.zeros((_NSLOT_PAD,), np.int32)
_q = 0
for _i in range(_F - 1):
    for _j in range(_i + 1, _F):
        _off_np[2 * _q] = _j * _ROWS_PER_TABLE + _i * _FIELD_DIM
        _fld_np[2 * _q] = _i
        _off_np[2 * _q + 1] = _i * _ROWS_PER_TABLE + _j * _FIELD_DIM
        _fld_np[2 * _q + 1] = _j
        _q += 1


@functools.cache
def _pack_kernel():
    info = plsc.get_sparse_core_info()
    nc, ns = info.num_cores, info.num_subcores
    nw = nc * ns
    nj = 64
    n_units = _F * nj
    mesh = plsc.VectorSubcoreMesh(core_axis_name="c", subcore_axis_name="s")

    @functools.partial(
        pl.kernel,
        mesh=mesh,
        out_type=jax.ShapeDtypeStruct((_NROW * _D,), jnp.float32),
        compiler_params=pltpu.CompilerParams(
            needs_layout_passes=False, use_tc_tiling_on_sc=True),
        scratch_types=[
            pltpu.VMEM((16, 128), jnp.float32),
            pltpu.VMEM((16, 128), jnp.float32),
            pltpu.VMEM((2048,), jnp.float32),
            pltpu.VMEM((2048,), jnp.float32),
            pltpu.VMEM((16, _W_TAIL), jnp.float32),
            pltpu.SemaphoreType.DMA,
            pltpu.SemaphoreType.DMA,
            pltpu.SemaphoreType.DMA,
            pltpu.SemaphoreType.DMA,
        ],
    )
    def k(tt_hbm, out_hbm, a0, a1, st0, st1, ta,
          semi0, semi1, semo0, semo1):
        wid = lax.axis_index("s") * nc + lax.axis_index("c")
        pos16 = lax.iota(jnp.int32, 16) * 16
        posb = [pos16 + dg for dg in range(16)]

        def unit_tr0(m):
            t = lax.shift_right_logical(m, 6)
            rb = jnp.minimum(wid + 32 * (m & 63), _RB_FULL - 1)
            r0 = pl.multiple_of(rb * 128, 128)
            return t, r0

        def fire_in(m, ba, sem):
            t, r0 = unit_tr0(m)
            pltpu.async_copy(
                tt_hbm.at[t, pl.ds(0, 16), pl.ds(r0, 128)], ba, sem)

        def wait_in(m, ba, sem):
            t, r0 = unit_tr0(m)
            pltpu.make_async_copy(
                tt_hbm.at[t, pl.ds(0, 16), pl.ds(r0, 128)], ba, sem).wait()

        def repack(ba, st):
            @plsc.parallel_loop(0, 128, 1, unroll=8)
            def _(i):
                d = lax.shift_right_logical(i, 3)
                c = i & 7
                c16 = pl.multiple_of(c * 16, 16)
                sc = c * 256 + d
                va = ba[d, pl.ds(c16, 16)]
                plsc.store_scatter(st, [pos16 + sc], va)

        def fire_out(m, st, sem):
            t, r0 = unit_tr0(m)
            base = pl.multiple_of((t * _ROWS_PER_TABLE + r0) * _D, 2048)
            pltpu.async_copy(st, out_hbm.at[pl.ds(base, 2048)], sem)

        def wait_out(m, st, sem):
            t, r0 = unit_tr0(m)
            base = pl.multiple_of((t * _ROWS_PER_TABLE + r0) * _D, 2048)
            pltpu.make_async_copy(st, out_hbm.at[pl.ds(base, 2048)], sem).wait()

        fire_in(jnp.int32(0), a0, semi0)

        def body(i, carry):
            m0 = 2 * i
            fire_in(m0 + 1, a1, semi1)
            wait_in(m0, a0, semi0)

            @pl.when(i > 0)
            def _():
                wait_out(m0 - 2, st0, semo0)

            repack(a0, st0)
            fire_out(m0, st0, semo0)

            @pl.when(i < n_units // 2 - 1)
            def _():
                fire_in(m0 + 2, a0, semi0)

            wait_in(m0 + 1, a1, semi1)

            @pl.when(i > 0)
            def _():
                wait_out(m0 - 1, st1, semo1)

            repack(a1, st1)
            fire_out(m0 + 1, st1, semo1)
            return carry

        lax.fori_loop(0, n_units // 2, body, 0)
        wait_out(jnp.int32(n_units - 2), st0, semo0)
        wait_out(jnp.int32(n_units - 1), st1, semo1)

        @pl.when(wid < _F)
        def _():
            t = wid
            pltpu.sync_copy(
                tt_hbm.at[t, pl.ds(0, 16), pl.ds(_R_TAIL, _W_TAIL)], ta)
            for d in range(16):
                for c in range(2):
                    va = ta[d, pl.ds(c * 16, 16)]
                    plsc.store_scatter(st0, [posb[d] + c * 256], va)
            base = (t * _ROWS_PER_TABLE + _R_TAIL) * _D
            pltpu.sync_copy(st0.at[pl.ds(0, 512)],
                            out_hbm.at[pl.ds(base, 512)])

    return k


@functools.cache
def _gather_kernel():
    info = plsc.get_sparse_core_info()
    nc, ns = info.num_cores, info.num_subcores
    nw = nc * ns
    assert _B % nw == 0
    bpw = _B // nw
    mesh = plsc.VectorSubcoreMesh(core_axis_name="c", subcore_axis_name="s")

    @functools.partial(
        pl.kernel,
        mesh=mesh,
        out_type=jax.ShapeDtypeStruct((_B,), jnp.float32),
        compiler_params=pltpu.CompilerParams(
            needs_layout_passes=False, use_tc_tiling_on_sc=False),
        scratch_types=[
            pltpu.VMEM((bpw * _F,), jnp.int32),
            pltpu.VMEM((_NSLOT_PAD,), jnp.int32),
            pltpu.VMEM((_NSLOT_PAD,), jnp.int32),
            pltpu.VMEM((_NSLOT_PAD,), jnp.int32),
            pltpu.VMEM((_NSLOT_PAD,), jnp.int32),
            pltpu.VMEM((_NSLOT_PAD, _D), jnp.float32),
            pltpu.VMEM((_NSLOT_PAD, _D), jnp.float32),
            pltpu.VMEM((bpw,), jnp.float32),
            pltpu.SemaphoreType.DMA,
            pltpu.SemaphoreType.DMA,
        ],
    )
    def k(tbl_hbm, x_hbm, off_hbm, fld_hbm, out_hbm,
          x_v, off_v, fld_v, idx0, idx1, rows0, rows1, out_v, sem0, sem1):
        wid = lax.axis_index("s") * nc + lax.axis_index("c")
        base = wid * bpw
        pltpu.sync_copy(x_hbm.at[pl.ds(base * _F, bpw * _F)], x_v)
        pltpu.sync_copy(off_hbm, off_v)
        pltpu.sync_copy(fld_hbm, fld_v)
        lane0 = lax.iota(jnp.int32, 16) == 0

        def fire(b, idx_v, rows_v, sem):
            b_vec = jnp.full((16,), b, jnp.int32)

            def build(c, bc):
                s0 = pl.multiple_of(c * 16, 16)
                off16 = off_v[pl.ds(s0, 16)]
                fld16 = fld_v[pl.ds(s0, 16)]
                xv = plsc.load_gather(x_v, [b_vec * _F + fld16])
                idx_v[pl.ds(s0, 16)] = off16 + xv
                return bc

            lax.fori_loop(0, _NSLOT_PAD // 16, build, 0, unroll=4)
            for st, ln in _CHUNKS:
                pltpu.async_copy(tbl_hbm.at[idx_v.at[pl.ds(st, ln)]],
                                 rows_v.at[pl.ds(st, ln)], sem)

        def drain(idx_v, rows_v, sem):
            for st, ln in _CHUNKS:
                pltpu.make_async_copy(tbl_hbm.at[idx_v.at[pl.ds(st, ln)]],
                                      rows_v.at[pl.ds(st, ln)], sem).wait()

        def compute(b, rows_v):
            def pair(p, acc):
                return acc + rows_v[2 * p] * rows_v[2 * p + 1]

            acc = lax.fori_loop(0, _NPAIR, pair,
                                jnp.zeros((16,), jnp.float32), unroll=8)
            sv = jnp.full((16,), jnp.sum(acc), jnp.float32)
            b_vec = jnp.full((16,), b, jnp.int32)
            plsc.store_scatter(out_v, [b_vec], sv, mask=lane0)

        fire(jnp.int32(0), idx0, rows0, sem0)

        def body(i, carry):
            b0 = 2 * i
            fire(b0 + 1, idx1, rows1, sem1)
            drain(idx0, rows0, sem0)
            compute(b0, rows0)

            @pl.when(i < bpw // 2 - 1)
            def _():
                fire(b0 + 2, idx0, rows0, sem0)

            drain(idx1, rows1, sem1)
            compute(b0 + 1, rows1)
            return carry

        lax.fori_loop(0, bpw // 2, body, 0)
        pltpu.sync_copy(out_v, out_hbm.at[pl.ds(base, bpw)])

    return k


def kernel(x, tables):
    x32 = x.astype(jnp.int32).reshape(-1)
    tt = tables.transpose(0, 2, 1)
    lin = _pack_kernel()(tt).reshape(_NROW, _D)
    off = jnp.asarray(_off_np)
    fld = jnp.asarray(_fld_np)
    out = _gather_kernel()(lin, x32, off, fld)
    return out.reshape(_B, 1)


# speedup vs baseline: 1.1515x; 1.1515x over previous
import functools

import numpy as np
import jax
import jax.numpy as jnp
from jax import lax
from jax.experimental import pallas as pl
from jax.experimental.pallas import tpu as pltpu
from jax.experimental.pallas import tpu_sc as plsc

_F = 26
_D = 16
_FIELD_DIM = 10000
_ROWS_PER_TABLE = _F * _FIELD_DIM
_NROW = _F * _ROWS_PER_TABLE
_B = 4096
_NPAIR = (_F * (_F - 1)) // 2
_NSLOT = 2 * _NPAIR
_NSLOT_PAD = 656
_CHUNKS = [(0, 128), (128, 128), (256, 128), (384, 128), (512, 128), (640, 16)]

_RB_FULL = _ROWS_PER_TABLE // 128
_R_TAIL = _RB_FULL * 128
_W_TAIL = _ROWS_PER_TABLE - _R_TAIL

_off_np = np.zeros((_NSLOT_PAD,), np.int32)
_fld_np = np.zeros((_NSLOT_PAD,), np.int32)
_q = 0
for _i in range(_F - 1):
    for _j in range(_i + 1, _F):
        _off_np[2 * _q] = _j * _ROWS_PER_TABLE + _i * _FIELD_DIM
        _fld_np[2 * _q] = _i
        _off_np[2 * _q + 1] = _i * _ROWS_PER_TABLE + _j * _FIELD_DIM
        _fld_np[2 * _q + 1] = _j
        _q += 1


@functools.cache
def _pack_kernel():
    info = plsc.get_sparse_core_info()
    nc, ns = info.num_cores, info.num_subcores
    nw = nc * ns
    nj = 64
    n_units = _F * nj
    mesh = plsc.VectorSubcoreMesh(core_axis_name="c", subcore_axis_name="s")

    @functools.partial(
        pl.kernel,
        mesh=mesh,
        out_type=jax.ShapeDtypeStruct((_NROW * _D,), jnp.float32),
        compiler_params=pltpu.CompilerParams(
            needs_layout_passes=False, use_tc_tiling_on_sc=True),
        scratch_types=[
            pltpu.VMEM((16, 128), jnp.float32),
            pltpu.VMEM((16, 128), jnp.float32),
            pltpu.VMEM((16, 128), jnp.float32),
            pltpu.VMEM((16, 128), jnp.float32),
            pltpu.VMEM((2048,), jnp.float32),
            pltpu.VMEM((2048,), jnp.float32),
            pltpu.VMEM((2048,), jnp.float32),
            pltpu.VMEM((2048,), jnp.float32),
            pltpu.VMEM((16, _W_TAIL), jnp.float32),
            pltpu.SemaphoreType.DMA,
            pltpu.SemaphoreType.DMA,
            pltpu.SemaphoreType.DMA,
            pltpu.SemaphoreType.DMA,
            pltpu.SemaphoreType.DMA,
            pltpu.SemaphoreType.DMA,
            pltpu.SemaphoreType.DMA,
            pltpu.SemaphoreType.DMA,
        ],
    )
    def k(tt_hbm, out_hbm, a0, a1, a2, a3, st0, st1, st2, st3, ta,
          semi0, semi1, semi2, semi3, semo0, semo1, semo2, semo3):
        wid = lax.axis_index("s") * nc + lax.axis_index("c")
        pos16 = lax.iota(jnp.int32, 16) * 16
        posb = [pos16 + dg for dg in range(16)]

        def unit_tr0(m):
            t = lax.shift_right_logical(m, 6)
            rb = jnp.minimum(wid + 32 * (m & 63), _RB_FULL - 1)
            r0 = pl.multiple_of(rb * 128, 128)
            return t, r0

        def fire_in(m, ba, sem):
            t, r0 = unit_tr0(m)
            pltpu.async_copy(
                tt_hbm.at[t, pl.ds(0, 16), pl.ds(r0, 128)], ba, sem)

        def wait_in(m, ba, sem):
            t, r0 = unit_tr0(m)
            pltpu.make_async_copy(
                tt_hbm.at[t, pl.ds(0, 16), pl.ds(r0, 128)], ba, sem).wait()

        def repack(ba, st):
            @plsc.parallel_loop(0, 128, 1, unroll=8)
            def _(i):
                d = lax.shift_right_logical(i, 3)
                c = i & 7
                c16 = pl.multiple_of(c * 16, 16)
                sc = c * 256 + d
                va = ba[d, pl.ds(c16, 16)]
                plsc.store_scatter(st, [pos16 + sc], va)

        def fire_out(m, st, sem):
            t, r0 = unit_tr0(m)
            base = pl.multiple_of((t * _ROWS_PER_TABLE + r0) * _D, 2048)
            pltpu.async_copy(st, out_hbm.at[pl.ds(base, 2048)], sem)

        def wait_out(m, st, sem):
            t, r0 = unit_tr0(m)
            base = pl.multiple_of((t * _ROWS_PER_TABLE + r0) * _D, 2048)
            pltpu.make_async_copy(st, out_hbm.at[pl.ds(base, 2048)], sem).wait()

        fire_in(jnp.int32(0), a0, semi0)
        fire_in(jnp.int32(1), a1, semi1)
        fire_in(jnp.int32(2), a2, semi2)
        fire_in(jnp.int32(3), a3, semi3)

        def body(i, carry):
            m0 = 4 * i
            for ph, (aP, stP, semiP, semoP) in enumerate(
                    [(a0, st0, semi0, semo0), (a1, st1, semi1, semo1),
                     (a2, st2, semi2, semo2), (a3, st3, semi3, semo3)]):
                wait_in(m0 + ph, aP, semiP)

                @pl.when(i > 0)
                def _():
                    wait_out(m0 + ph - 4, stP, semoP)

                repack(aP, stP)
                fire_out(m0 + ph, stP, semoP)

                @pl.when(i < n_units // 4 - 1)
                def _():
                    fire_in(m0 + ph + 4, aP, semiP)
            return carry

        lax.fori_loop(0, n_units // 4, body, 0)
        wait_out(jnp.int32(n_units - 4), st0, semo0)
        wait_out(jnp.int32(n_units - 3), st1, semo1)
        wait_out(jnp.int32(n_units - 2), st2, semo2)
        wait_out(jnp.int32(n_units - 1), st3, semo3)

        @pl.when(wid < _F)
        def _():
            t = wid
            pltpu.sync_copy(
                tt_hbm.at[t, pl.ds(0, 16), pl.ds(_R_TAIL, _W_TAIL)], ta)
            for d in range(16):
                for c in range(2):
                    va = ta[d, pl.ds(c * 16, 16)]
                    plsc.store_scatter(st0, [posb[d] + c * 256], va)
            base = (t * _ROWS_PER_TABLE + _R_TAIL) * _D
            pltpu.sync_copy(st0.at[pl.ds(0, 512)],
                            out_hbm.at[pl.ds(base, 512)])

    return k


@functools.cache
def _gather_kernel():
    info = plsc.get_sparse_core_info()
    nc, ns = info.num_cores, info.num_subcores
    nw = nc * ns
    assert _B % nw == 0
    bpw = _B // nw
    mesh = plsc.VectorSubcoreMesh(core_axis_name="c", subcore_axis_name="s")

    @functools.partial(
        pl.kernel,
        mesh=mesh,
        out_type=jax.ShapeDtypeStruct((_B,), jnp.float32),
        compiler_params=pltpu.CompilerParams(
            needs_layout_passes=False, use_tc_tiling_on_sc=False),
        scratch_types=[
            pltpu.VMEM((bpw * _F,), jnp.int32),
            pltpu.VMEM((_NSLOT_PAD,), jnp.int32),
            pltpu.VMEM((_NSLOT_PAD,), jnp.int32),
            pltpu.VMEM((_NSLOT_PAD,), jnp.int32),
            pltpu.VMEM((_NSLOT_PAD,), jnp.int32),
            pltpu.VMEM((_NSLOT_PAD, _D), jnp.float32),
            pltpu.VMEM((_NSLOT_PAD, _D), jnp.float32),
            pltpu.VMEM((bpw,), jnp.float32),
            pltpu.SemaphoreType.DMA,
            pltpu.SemaphoreType.DMA,
        ],
    )
    def k(tbl_hbm, x_hbm, off_hbm, fld_hbm, out_hbm,
          x_v, off_v, fld_v, idx0, idx1, rows0, rows1, out_v, sem0, sem1):
        wid = lax.axis_index("s") * nc + lax.axis_index("c")
        base = wid * bpw
        pltpu.sync_copy(x_hbm.at[pl.ds(base * _F, bpw * _F)], x_v)
        pltpu.sync_copy(off_hbm, off_v)
        pltpu.sync_copy(fld_hbm, fld_v)
        lane0 = lax.iota(jnp.int32, 16) == 0

        def fire(b, idx_v, rows_v, sem):
            b_vec = jnp.full((16,), b, jnp.int32)

            def build(c, bc):
                s0 = pl.multiple_of(c * 16, 16)
                off16 = off_v[pl.ds(s0, 16)]
                fld16 = fld_v[pl.ds(s0, 16)]
                xv = plsc.load_gather(x_v, [b_vec * _F + fld16])
                idx_v[pl.ds(s0, 16)] = off16 + xv
                return bc

            lax.fori_loop(0, _NSLOT_PAD // 16, build, 0, unroll=4)
            for st, ln in _CHUNKS:
                pltpu.async_copy(tbl_hbm.at[idx_v.at[pl.ds(st, ln)]],
                                 rows_v.at[pl.ds(st, ln)], sem)

        def drain(idx_v, rows_v, sem):
            for st, ln in _CHUNKS:
                pltpu.make_async_copy(tbl_hbm.at[idx_v.at[pl.ds(st, ln)]],
                                      rows_v.at[pl.ds(st, ln)], sem).wait()

        def compute(b, rows_v):
            def pair(p, acc):
                return acc + rows_v[2 * p] * rows_v[2 * p + 1]

            acc = lax.fori_loop(0, _NPAIR, pair,
                                jnp.zeros((16,), jnp.float32), unroll=8)
            sv = jnp.full((16,), jnp.sum(acc), jnp.float32)
            b_vec = jnp.full((16,), b, jnp.int32)
            plsc.store_scatter(out_v, [b_vec], sv, mask=lane0)

        fire(jnp.int32(0), idx0, rows0, sem0)

        def body(i, carry):
            b0 = 2 * i
            fire(b0 + 1, idx1, rows1, sem1)
            drain(idx0, rows0, sem0)
            compute(b0, rows0)

            @pl.when(i < bpw // 2 - 1)
            def _():
                fire(b0 + 2, idx0, rows0, sem0)

            drain(idx1, rows1, sem1)
            compute(b0 + 1, rows1)
            return carry

        lax.fori_loop(0, bpw // 2, body, 0)
        pltpu.sync_copy(out_v, out_hbm.at[pl.ds(base, bpw)])

    return k


def kernel(x, tables):
    x32 = x.astype(jnp.int32).reshape(-1)
    tt = tables.transpose(0, 2, 1)
    lin = _pack_kernel()(tt).reshape(_NROW, _D)
    off = jnp.asarray(_off_np)
    fld = jnp.asarray(_fld_np)
    out = _gather_kernel()(lin, x32, off, fld)
    return out.reshape(_B, 1)
